# baseline (speedup 1.0000x reference)
import functools

import jax
import jax.numpy as jnp
from jax import lax
from jax.experimental import pallas as pl
from jax.experimental.pallas import tpu as pltpu
from jax.experimental.pallas import tpu_sc as plsc

_ZLOSS_COEF = 1e-3
_NEG = -1e30
_TILE = 256
_GW = 128


def _router_kernel(x_ref, wd_ref, gw_ref, bias_ref,
                   z_ref, sel_ref, gat_ref, pos_ref, te_ref, zl_ref,
                   carry_ref, csel_ref):
    i = pl.program_id(0)
    n = pl.num_programs(0)
    blk = x_ref.shape[0]
    row0 = i * blk
    x = x_ref[...]
    z = jnp.dot(x, wd_ref[...], preferred_element_type=jnp.float32)
    z_ref[...] = z
    logits = jnp.dot(z, gw_ref[...], preferred_element_type=jnp.float32)
    m = jnp.max(logits, axis=1, keepdims=True)
    lse = m[:, 0] + jnp.log(jnp.sum(jnp.exp(logits - m), axis=1))
    part = jnp.sum(lse * lse)

    @pl.when(i == 0)
    def _():
        zl_ref[...] = part.reshape(1, 1)

    @pl.when(i != 0)
    def _():
        zl_ref[...] += part.reshape(1, 1)

    aff = jax.nn.sigmoid(logits)
    scores = aff + bias_ref[...]
    E = scores.shape[1]
    iota = lax.broadcasted_iota(jnp.int32, scores.shape, 1)
    m1 = jnp.max(scores, axis=1, keepdims=True)
    a1 = jnp.min(jnp.where(scores == m1, iota, E), axis=1)
    oh1 = iota == a1[:, None]
    scores2 = jnp.where(oh1, _NEG, scores)
    m2 = jnp.max(scores2, axis=1, keepdims=True)
    a2 = jnp.min(jnp.where(scores2 == m2, iota, E), axis=1)
    oh2 = iota == a2[:, None]
    aff1 = jnp.sum(jnp.where(oh1, aff, 0.0), axis=1)
    aff2 = jnp.sum(jnp.where(oh2, aff, 0.0), axis=1)
    denom = aff1 + aff2 + 1e-9
    g1 = aff1 / denom
    g2 = aff2 / denom
    sel_ref[pl.ds(row0, blk), :] = jnp.concatenate(
        [a1[:, None], a2[:, None]], axis=1)
    gat_ref[...] = jnp.concatenate([g1[:, None], g2[:, None]], axis=1)

    A = oh1.astype(jnp.float32) + oh2.astype(jnp.float32)
    r = lax.broadcasted_iota(jnp.int32, (blk, blk), 0)
    c = lax.broadcasted_iota(jnp.int32, (blk, blk), 1)
    tril = (r >= c).astype(jnp.float32)
    cin = jnp.dot(tril, A, preferred_element_type=jnp.float32)

    @pl.when(i == 0)
    def _():
        carry_ref[...] = jnp.zeros_like(carry_ref)

    carry = carry_ref[...]
    cex = cin - A + carry
    csel0 = jnp.sum(jnp.where(oh1, cex, 0.0), axis=1)
    csel1 = jnp.sum(jnp.where(oh2, cex, 0.0), axis=1)
    csel_ref[pl.ds(row0, blk), :] = jnp.concatenate(
        [csel0[:, None], csel1[:, None]], axis=1)
    newcarry = carry + jnp.sum(A, axis=0, keepdims=True)
    carry_ref[...] = newcarry

    @pl.when(i == n - 1)
    def _():
        counts = newcarry
        pc = jnp.floor((counts + (_TILE - 1.0))
                       * (1.0 / _TILE)) * _TILE
        rr = lax.broadcasted_iota(jnp.int32, (E, E), 0)
        cc = lax.broadcasted_iota(jnp.int32, (E, E), 1)
        U = (rr < cc).astype(jnp.float32)
        pc8 = jnp.broadcast_to(pc, (8, E))
        cumx = jnp.dot(pc8, U, preferred_element_type=jnp.float32,
                       precision=jax.lax.Precision.HIGHEST)[0:1]
        cumi = cumx + pc
        sel_all = sel_ref[...]
        csel_all = csel_ref[...]
        T = sel_all.shape[0]
        eio = lax.broadcasted_iota(jnp.int32, (T, E), 1)
        cumxB = jnp.broadcast_to(cumx, (T, E))
        p0 = jnp.sum(jnp.where(eio == sel_all[:, 0][:, None], cumxB, 0.0),
                     axis=1) + csel_all[:, 0]
        p1 = jnp.sum(jnp.where(eio == sel_all[:, 1][:, None], cumxB, 0.0),
                     axis=1) + csel_all[:, 1]
        pos_ref[...] = jnp.concatenate(
            [p0[:, None], p1[:, None]], axis=1).astype(jnp.int32)

        TE_ROWS = te_ref.shape[0]
        NT = TE_ROWS - 16
        r64 = lax.broadcasted_iota(jnp.int32, (TE_ROWS, 1), 0)
        tstart = r64.astype(jnp.float32) * float(_TILE)
        cumiB = jnp.broadcast_to(cumi, (TE_ROWS, E))
        te_v = jnp.sum((cumiB <= tstart).astype(jnp.float32), axis=1,
                       keepdims=True)
        te_v = jnp.minimum(te_v, float(E - 1))
        n_act = cumi[:, E - 1:E] * (1.0 / _TILE)
        te_v = jnp.where(r64 == NT, jnp.broadcast_to(n_act, (TE_ROWS, 1)),
                         te_v)
        te_ref[...] = te_v.astype(jnp.int32)


def _expert_kernel(te_ref, zg_ref, guw_ref, dw_ref, eo_ref):
    i = pl.program_id(0)
    n_act = te_ref[te_ref.shape[0] - 16, 0]

    @pl.when(i < n_act)
    def _():
        zt = zg_ref[...].astype(jnp.bfloat16)
        h = jnp.dot(zt, guw_ref[0].astype(jnp.bfloat16),
                    preferred_element_type=jnp.float32)
        F = h.shape[1] // 2
        hh = (jax.nn.silu(h[:, :F]) * h[:, F:]).astype(jnp.bfloat16)
        eo = jnp.dot(hh, dw_ref[0].astype(jnp.bfloat16),
                     preferred_element_type=jnp.float32)
        eo_ref[...] = eo


def _shared_kernel(x_ref, sguw_ref, sdw_ref, so_ref,
                   sguw16_ref, sdw16_ref):
    @pl.when(pl.program_id(0) == 0)
    def _():
        sguw16_ref[...] = sguw_ref[...].astype(jnp.bfloat16)
        sdw16_ref[...] = sdw_ref[...].astype(jnp.bfloat16)

    x = x_ref[...].astype(jnp.bfloat16)
    Fs = sdw_ref.shape[0]
    CH = 512
    acc = jnp.zeros((x.shape[0], sdw_ref.shape[1]), jnp.float32)
    for j in range(Fs // CH):
        gj = jnp.dot(x, sguw16_ref[:, j * CH:(j + 1) * CH],
                     preferred_element_type=jnp.float32)
        uj = jnp.dot(x, sguw16_ref[:, Fs + j * CH:Fs + (j + 1) * CH],
                     preferred_element_type=jnp.float32)
        hh = (jax.nn.silu(gj) * uj).astype(jnp.bfloat16)
        acc = acc + jnp.dot(hh, sdw16_ref[j * CH:(j + 1) * CH, :],
                            preferred_element_type=jnp.float32)
    so_ref[...] = acc


def _combine_kernel(so_ref, r0_ref, r1_ref, gat_ref, wup_ref, out_ref):
    g0 = gat_ref[:, 0:1]
    g1 = gat_ref[:, 1:2]
    ol = (g0 * r0_ref[...] + g1 * r1_ref[...]).astype(jnp.bfloat16)
    out_ref[...] = so_ref[...] + jnp.dot(
        ol, wup_ref[...].astype(jnp.bfloat16),
        preferred_element_type=jnp.float32)


def _sc_dispatch(zl_, pos0, pos1, pad_rows):
    T, L = zl_.shape
    mesh = plsc.VectorSubcoreMesh(core_axis_name="c", subcore_axis_name="s")
    info = plsc.get_sparse_core_info()
    NC, NS = info.num_cores, info.num_subcores
    chunk = T // (NC * NS)

    @functools.partial(
        pl.kernel, mesh=mesh,
        out_type=jax.ShapeDtypeStruct((pad_rows, L), zl_.dtype),
        scratch_types=[
            pltpu.VMEM((chunk,), jnp.int32),
            pltpu.VMEM((chunk,), jnp.int32),
            pltpu.VMEM((chunk, L), zl_.dtype),
            pltpu.SemaphoreType.DMA,
            pltpu.SemaphoreType.DMA,
        ])
    def k(z_hbm, p0_hbm, p1_hbm, zg_hbm, p0_v, p1_v, rows_v, sem1, sem2):
        wid = lax.axis_index("s") * NC + lax.axis_index("c")
        base = wid * chunk
        pltpu.sync_copy(p0_hbm.at[pl.ds(base, chunk)], p0_v)
        pltpu.sync_copy(p1_hbm.at[pl.ds(base, chunk)], p1_v)
        pltpu.sync_copy(z_hbm.at[pl.ds(base, chunk)], rows_v)
        c1 = pltpu.async_copy(rows_v, zg_hbm.at[p0_v], sem1)
        c2 = pltpu.async_copy(rows_v, zg_hbm.at[p1_v], sem2)
        c1.wait()
        c2.wait()

    return k(zl_, pos0, pos1)


def _sc_combine(eo_, pos0, pos1):
    _, L = eo_.shape
    T = pos0.shape[0]
    mesh = plsc.VectorSubcoreMesh(core_axis_name="c", subcore_axis_name="s")
    info = plsc.get_sparse_core_info()
    NC, NS = info.num_cores, info.num_subcores
    chunk = T // (NC * NS)

    @functools.partial(
        pl.kernel, mesh=mesh,
        out_type=(jax.ShapeDtypeStruct((T, L), eo_.dtype),
                  jax.ShapeDtypeStruct((T, L), eo_.dtype)),
        scratch_types=[
            pltpu.VMEM((chunk,), jnp.int32),
            pltpu.VMEM((chunk,), jnp.int32),
            pltpu.VMEM((chunk, L), eo_.dtype),
            pltpu.VMEM((chunk, L), eo_.dtype),
            pltpu.SemaphoreType.DMA,
            pltpu.SemaphoreType.DMA,
        ])
    def k(eo_hbm, p0_hbm, p1_hbm, r0_hbm, r1_hbm,
          p0_v, p1_v, r0_v, r1_v, sem1, sem2):
        wid = lax.axis_index("s") * NC + lax.axis_index("c")
        base = wid * chunk
        pltpu.sync_copy(p0_hbm.at[pl.ds(base, chunk)], p0_v)
        pltpu.sync_copy(p1_hbm.at[pl.ds(base, chunk)], p1_v)
        c1 = pltpu.async_copy(eo_hbm.at[p0_v], r0_v, sem1)
        c2 = pltpu.async_copy(eo_hbm.at[p1_v], r1_v, sem2)
        c1.wait()
        c2.wait()
        pltpu.sync_copy(r0_v, r0_hbm.at[pl.ds(base, chunk)])
        pltpu.sync_copy(r1_v, r1_hbm.at[pl.ds(base, chunk)])

    return k(eo_, pos0, pos1)


def kernel(x, W_down, gate_w, expert_bias, gate_up_w, down_w, W_up,
           shared_gu_w, shared_down_w):
    orig_shape = x.shape
    D = x.shape[-1]
    x_flat = x.reshape(-1, D)
    T = x_flat.shape[0]
    L = W_down.shape[1]
    E = gate_w.shape[1]
    F = down_w.shape[1]
    Fs = shared_down_w.shape[0]
    TOPK = 2
    NP = T * TOPK
    PAD = ((NP + E * (_TILE - 1) + _TILE - 1) // _TILE) * _TILE
    NT = PAD // _TILE

    blk = 512
    grid_t = T // blk
    fblk = 512
    fgrid = T // fblk

    zrt, selected, gating, pos, te, zl = pl.pallas_call(
        _router_kernel,
        grid=(grid_t,),
        in_specs=[
            pl.BlockSpec((blk, D), lambda i: (i, 0)),
            pl.BlockSpec((D, L), lambda i: (0, 0)),
            pl.BlockSpec((L, E), lambda i: (0, 0)),
            pl.BlockSpec((1, E), lambda i: (0, 0)),
        ],
        out_specs=[
            pl.BlockSpec((blk, L), lambda i: (i, 0)),
            pl.BlockSpec((T, 2), lambda i: (0, 0)),
            pl.BlockSpec((blk, 2), lambda i: (i, 0)),
            pl.BlockSpec((T, 2), lambda i: (0, 0)),
            pl.BlockSpec((NT + 16, 1), lambda i: (0, 0)),
            pl.BlockSpec((1, 1), lambda i: (0, 0)),
        ],
        out_shape=[
            jax.ShapeDtypeStruct((T, L), jnp.float32),
            jax.ShapeDtypeStruct((T, 2), jnp.int32),
            jax.ShapeDtypeStruct((T, 2), jnp.float32),
            jax.ShapeDtypeStruct((T, 2), jnp.int32),
            jax.ShapeDtypeStruct((NT + 16, 1), jnp.int32),
            jax.ShapeDtypeStruct((1, 1), jnp.float32),
        ],
        scratch_shapes=[pltpu.VMEM((1, E), jnp.float32),
                        pltpu.VMEM((T, 2), jnp.float32)],
    )(x_flat, W_down, gate_w, expert_bias.reshape(1, E))

    z_loss = (_ZLOSS_COEF / T) * zl[0, 0]

    pos0 = pos[:, 0]
    pos1 = pos[:, 1]

    zg = _sc_dispatch(zrt, pos0, pos1, PAD)

    grid_spec = pltpu.PrefetchScalarGridSpec(
        num_scalar_prefetch=1,
        grid=(NT,),
        in_specs=[
            pl.BlockSpec((_TILE, L), lambda i, te_r: (i, 0)),
            pl.BlockSpec((1, L, 2 * F), lambda i, te_r: (te_r[i, 0], 0, 0)),
            pl.BlockSpec((1, F, L), lambda i, te_r: (te_r[i, 0], 0, 0)),
        ],
        out_specs=pl.BlockSpec((_TILE, L), lambda i, te_r: (i, 0)),
    )
    eo = pl.pallas_call(
        _expert_kernel,
        grid_spec=grid_spec,
        out_shape=jax.ShapeDtypeStruct((PAD, L), jnp.float32),
        compiler_params=pltpu.CompilerParams(
            dimension_semantics=("arbitrary",)),
    )(te, zg, gate_up_w, down_w)

    r0, r1 = _sc_combine(eo, pos0, pos1)

    so = pl.pallas_call(
        _shared_kernel,
        grid=(fgrid,),
        in_specs=[
            pl.BlockSpec((fblk, D), lambda i: (i, 0)),
            pl.BlockSpec((D, 2 * Fs), lambda i: (0, 0)),
            pl.BlockSpec((Fs, D), lambda i: (0, 0)),
        ],
        out_specs=pl.BlockSpec((fblk, D), lambda i: (i, 0)),
        out_shape=jax.ShapeDtypeStruct((T, D), jnp.float32),
        scratch_shapes=[pltpu.VMEM((D, 2 * Fs), jnp.bfloat16),
                        pltpu.VMEM((Fs, D), jnp.bfloat16)],
    )(x_flat, shared_gu_w, shared_down_w)

    out = pl.pallas_call(
        _combine_kernel,
        grid=(fgrid,),
        in_specs=[
            pl.BlockSpec((fblk, D), lambda i: (i, 0)),
            pl.BlockSpec((fblk, L), lambda i: (i, 0)),
            pl.BlockSpec((fblk, L), lambda i: (i, 0)),
            pl.BlockSpec((fblk, 2), lambda i: (i, 0)),
            pl.BlockSpec((L, D), lambda i: (0, 0)),
        ],
        out_specs=pl.BlockSpec((fblk, D), lambda i: (i, 0)),
        out_shape=jax.ShapeDtypeStruct((T, D), jnp.float32),
    )(so, r0, r1, gating, W_up)

    return (out.reshape(orig_shape), selected, gating, z_loss)


# speedup vs baseline: 1.0643x; 1.0643x over previous
import functools

import jax
import jax.numpy as jnp
from jax import lax
from jax.experimental import pallas as pl
from jax.experimental.pallas import tpu as pltpu
from jax.experimental.pallas import tpu_sc as plsc

_ZLOSS_COEF = 1e-3
_NEG = -1e30
_TILE = 256
_GW = 128


def _router_kernel(x_ref, wd_ref, gw_ref, bias_ref,
                   z_ref, sel_ref, gat_ref, pos_ref, te_ref, zl_ref,
                   carry_ref, csel_ref):
    i = pl.program_id(0)
    n = pl.num_programs(0)
    blk = x_ref.shape[0]
    row0 = i * blk
    x = x_ref[...].astype(jnp.bfloat16)
    z = jnp.dot(x, wd_ref[...].astype(jnp.bfloat16),
                preferred_element_type=jnp.float32)
    z_ref[...] = z
    logits = jnp.dot(z, gw_ref[...], preferred_element_type=jnp.float32)
    m = jnp.max(logits, axis=1, keepdims=True)
    lse = m[:, 0] + jnp.log(jnp.sum(jnp.exp(logits - m), axis=1))
    part = jnp.sum(lse * lse)

    @pl.when(i == 0)
    def _():
        zl_ref[...] = part.reshape(1, 1)

    @pl.when(i != 0)
    def _():
        zl_ref[...] += part.reshape(1, 1)

    aff = jax.nn.sigmoid(logits)
    scores = aff + bias_ref[...]
    E = scores.shape[1]
    iota = lax.broadcasted_iota(jnp.int32, scores.shape, 1)
    m1 = jnp.max(scores, axis=1, keepdims=True)
    a1 = jnp.min(jnp.where(scores == m1, iota, E), axis=1)
    oh1 = iota == a1[:, None]
    scores2 = jnp.where(oh1, _NEG, scores)
    m2 = jnp.max(scores2, axis=1, keepdims=True)
    a2 = jnp.min(jnp.where(scores2 == m2, iota, E), axis=1)
    oh2 = iota == a2[:, None]
    aff1 = jnp.sum(jnp.where(oh1, aff, 0.0), axis=1)
    aff2 = jnp.sum(jnp.where(oh2, aff, 0.0), axis=1)
    denom = aff1 + aff2 + 1e-9
    g1 = aff1 / denom
    g2 = aff2 / denom
    sel_ref[pl.ds(row0, blk), :] = jnp.concatenate(
        [a1[:, None], a2[:, None]], axis=1)
    gat_ref[...] = jnp.concatenate([g1[:, None], g2[:, None]], axis=1)

    A = oh1.astype(jnp.float32) + oh2.astype(jnp.float32)
    r = lax.broadcasted_iota(jnp.int32, (blk, blk), 0)
    c = lax.broadcasted_iota(jnp.int32, (blk, blk), 1)
    tril = (r >= c).astype(jnp.float32)
    cin = jnp.dot(tril, A, preferred_element_type=jnp.float32)

    @pl.when(i == 0)
    def _():
        carry_ref[...] = jnp.zeros_like(carry_ref)

    carry = carry_ref[...]
    cex = cin - A + carry
    csel0 = jnp.sum(jnp.where(oh1, cex, 0.0), axis=1)
    csel1 = jnp.sum(jnp.where(oh2, cex, 0.0), axis=1)
    csel_ref[pl.ds(row0, blk), :] = jnp.concatenate(
        [csel0[:, None], csel1[:, None]], axis=1)
    newcarry = carry + jnp.sum(A, axis=0, keepdims=True)
    carry_ref[...] = newcarry

    @pl.when(i == n - 1)
    def _():
        counts = newcarry
        pc = jnp.floor((counts + (_TILE - 1.0))
                       * (1.0 / _TILE)) * _TILE
        rr = lax.broadcasted_iota(jnp.int32, (E, E), 0)
        cc = lax.broadcasted_iota(jnp.int32, (E, E), 1)
        U = (rr < cc).astype(jnp.float32)
        pc8 = jnp.broadcast_to(pc, (8, E))
        cumx = jnp.dot(pc8, U, preferred_element_type=jnp.float32,
                       precision=jax.lax.Precision.HIGHEST)[0:1]
        cumi = cumx + pc
        sel_all = sel_ref[...]
        csel_all = csel_ref[...]
        T = sel_all.shape[0]
        eio = lax.broadcasted_iota(jnp.int32, (T, E), 1)
        cumxB = jnp.broadcast_to(cumx, (T, E))
        p0 = jnp.sum(jnp.where(eio == sel_all[:, 0][:, None], cumxB, 0.0),
                     axis=1) + csel_all[:, 0]
        p1 = jnp.sum(jnp.where(eio == sel_all[:, 1][:, None], cumxB, 0.0),
                     axis=1) + csel_all[:, 1]
        pos_ref[...] = jnp.concatenate(
            [p0[:, None], p1[:, None]], axis=1).astype(jnp.int32)

        TE_ROWS = te_ref.shape[0]
        NT = TE_ROWS - 16
        r64 = lax.broadcasted_iota(jnp.int32, (TE_ROWS, 1), 0)
        tstart = r64.astype(jnp.float32) * float(_TILE)
        cumiB = jnp.broadcast_to(cumi, (TE_ROWS, E))
        te_v = jnp.sum((cumiB <= tstart).astype(jnp.float32), axis=1,
                       keepdims=True)
        te_v = jnp.minimum(te_v, float(E - 1))
        n_act = cumi[:, E - 1:E] * (1.0 / _TILE)
        te_v = jnp.where(r64 == NT, jnp.broadcast_to(n_act, (TE_ROWS, 1)),
                         te_v)
        te_ref[...] = te_v.astype(jnp.int32)


def _expert_kernel(te_ref, zg_ref, guw_ref, dw_ref, eo_ref):
    i = pl.program_id(0)
    n_act = te_ref[te_ref.shape[0] - 16, 0]

    @pl.when(i < n_act)
    def _():
        zt = zg_ref[...].astype(jnp.bfloat16)
        h = jnp.dot(zt, guw_ref[0].astype(jnp.bfloat16),
                    preferred_element_type=jnp.float32)
        F = h.shape[1] // 2
        hh = (jax.nn.silu(h[:, :F]) * h[:, F:]).astype(jnp.bfloat16)
        eo = jnp.dot(hh, dw_ref[0].astype(jnp.bfloat16),
                     preferred_element_type=jnp.float32)
        eo_ref[...] = eo


def _final_kernel(x_ref, r0_ref, r1_ref, gat_ref, sguw_ref, sdw_ref, wup_ref,
                  out_ref, sguw16_ref, sdw16_ref, wup16_ref):
    @pl.when(pl.program_id(0) == 0)
    def _():
        sguw16_ref[...] = sguw_ref[...].astype(jnp.bfloat16)
        sdw16_ref[...] = sdw_ref[...].astype(jnp.bfloat16)
        wup16_ref[...] = wup_ref[...].astype(jnp.bfloat16)

    x = x_ref[...].astype(jnp.bfloat16)
    Fs = sdw_ref.shape[0]
    CH = 512
    g0 = gat_ref[:, 0:1]
    g1 = gat_ref[:, 1:2]
    ol = (g0 * r0_ref[...] + g1 * r1_ref[...]).astype(jnp.bfloat16)
    acc = jnp.dot(ol, wup16_ref[...], preferred_element_type=jnp.float32)
    for j in range(Fs // CH):
        gj = jnp.dot(x, sguw16_ref[:, j * CH:(j + 1) * CH],
                     preferred_element_type=jnp.float32)
        uj = jnp.dot(x, sguw16_ref[:, Fs + j * CH:Fs + (j + 1) * CH],
                     preferred_element_type=jnp.float32)
        hh = (jax.nn.silu(gj) * uj).astype(jnp.bfloat16)
        acc = acc + jnp.dot(hh, sdw16_ref[j * CH:(j + 1) * CH, :],
                            preferred_element_type=jnp.float32)
    out_ref[...] = acc


def _sc_dispatch(zl_, pos0, pos1, pad_rows):
    T, L = zl_.shape
    mesh = plsc.VectorSubcoreMesh(core_axis_name="c", subcore_axis_name="s")
    info = plsc.get_sparse_core_info()
    NC, NS = info.num_cores, info.num_subcores
    chunk = T // (NC * NS)

    @functools.partial(
        pl.kernel, mesh=mesh,
        out_type=jax.ShapeDtypeStruct((pad_rows, L), zl_.dtype),
        scratch_types=[
            pltpu.VMEM((chunk,), jnp.int32),
            pltpu.VMEM((chunk,), jnp.int32),
            pltpu.VMEM((chunk, L), zl_.dtype),
            pltpu.SemaphoreType.DMA,
            pltpu.SemaphoreType.DMA,
        ])
    def k(z_hbm, p0_hbm, p1_hbm, zg_hbm, p0_v, p1_v, rows_v, sem1, sem2):
        wid = lax.axis_index("s") * NC + lax.axis_index("c")
        base = wid * chunk
        pltpu.sync_copy(p0_hbm.at[pl.ds(base, chunk)], p0_v)
        pltpu.sync_copy(p1_hbm.at[pl.ds(base, chunk)], p1_v)
        pltpu.sync_copy(z_hbm.at[pl.ds(base, chunk)], rows_v)
        c1 = pltpu.async_copy(rows_v, zg_hbm.at[p0_v], sem1)
        c2 = pltpu.async_copy(rows_v, zg_hbm.at[p1_v], sem2)
        c1.wait()
        c2.wait()

    return k(zl_, pos0, pos1)


def _sc_combine(eo_, pos0, pos1):
    _, L = eo_.shape
    T = pos0.shape[0]
    mesh = plsc.VectorSubcoreMesh(core_axis_name="c", subcore_axis_name="s")
    info = plsc.get_sparse_core_info()
    NC, NS = info.num_cores, info.num_subcores
    chunk = T // (NC * NS)

    @functools.partial(
        pl.kernel, mesh=mesh,
        out_type=(jax.ShapeDtypeStruct((T, L), eo_.dtype),
                  jax.ShapeDtypeStruct((T, L), eo_.dtype)),
        scratch_types=[
            pltpu.VMEM((chunk,), jnp.int32),
            pltpu.VMEM((chunk,), jnp.int32),
            pltpu.VMEM((chunk, L), eo_.dtype),
            pltpu.VMEM((chunk, L), eo_.dtype),
            pltpu.SemaphoreType.DMA,
            pltpu.SemaphoreType.DMA,
        ])
    def k(eo_hbm, p0_hbm, p1_hbm, r0_hbm, r1_hbm,
          p0_v, p1_v, r0_v, r1_v, sem1, sem2):
        wid = lax.axis_index("s") * NC + lax.axis_index("c")
        base = wid * chunk
        pltpu.sync_copy(p0_hbm.at[pl.ds(base, chunk)], p0_v)
        pltpu.sync_copy(p1_hbm.at[pl.ds(base, chunk)], p1_v)
        c1 = pltpu.async_copy(eo_hbm.at[p0_v], r0_v, sem1)
        c2 = pltpu.async_copy(eo_hbm.at[p1_v], r1_v, sem2)
        c1.wait()
        c2.wait()
        pltpu.sync_copy(r0_v, r0_hbm.at[pl.ds(base, chunk)])
        pltpu.sync_copy(r1_v, r1_hbm.at[pl.ds(base, chunk)])

    return k(eo_, pos0, pos1)


def kernel(x, W_down, gate_w, expert_bias, gate_up_w, down_w, W_up,
           shared_gu_w, shared_down_w):
    orig_shape = x.shape
    D = x.shape[-1]
    x_flat = x.reshape(-1, D)
    T = x_flat.shape[0]
    L = W_down.shape[1]
    E = gate_w.shape[1]
    F = down_w.shape[1]
    Fs = shared_down_w.shape[0]
    TOPK = 2
    NP = T * TOPK
    PAD = ((NP + E * (_TILE - 1) + _TILE - 1) // _TILE) * _TILE
    NT = PAD // _TILE

    blk = 512
    grid_t = T // blk
    fblk = 512
    fgrid = T // fblk

    zrt, selected, gating, pos, te, zl = pl.pallas_call(
        _router_kernel,
        grid=(grid_t,),
        in_specs=[
            pl.BlockSpec((blk, D), lambda i: (i, 0)),
            pl.BlockSpec((D, L), lambda i: (0, 0)),
            pl.BlockSpec((L, E), lambda i: (0, 0)),
            pl.BlockSpec((1, E), lambda i: (0, 0)),
        ],
        out_specs=[
            pl.BlockSpec((blk, L), lambda i: (i, 0)),
            pl.BlockSpec((T, 2), lambda i: (0, 0)),
            pl.BlockSpec((blk, 2), lambda i: (i, 0)),
            pl.BlockSpec((T, 2), lambda i: (0, 0)),
            pl.BlockSpec((NT + 16, 1), lambda i: (0, 0)),
            pl.BlockSpec((1, 1), lambda i: (0, 0)),
        ],
        out_shape=[
            jax.ShapeDtypeStruct((T, L), jnp.float32),
            jax.ShapeDtypeStruct((T, 2), jnp.int32),
            jax.ShapeDtypeStruct((T, 2), jnp.float32),
            jax.ShapeDtypeStruct((T, 2), jnp.int32),
            jax.ShapeDtypeStruct((NT + 16, 1), jnp.int32),
            jax.ShapeDtypeStruct((1, 1), jnp.float32),
        ],
        scratch_shapes=[pltpu.VMEM((1, E), jnp.float32),
                        pltpu.VMEM((T, 2), jnp.float32)],
    )(x_flat, W_down, gate_w, expert_bias.reshape(1, E))

    z_loss = (_ZLOSS_COEF / T) * zl[0, 0]

    pos0 = pos[:, 0]
    pos1 = pos[:, 1]

    zg = _sc_dispatch(zrt, pos0, pos1, PAD)

    grid_spec = pltpu.PrefetchScalarGridSpec(
        num_scalar_prefetch=1,
        grid=(NT,),
        in_specs=[
            pl.BlockSpec((_TILE, L), lambda i, te_r: (i, 0)),
            pl.BlockSpec((1, L, 2 * F), lambda i, te_r: (te_r[i, 0], 0, 0)),
            pl.BlockSpec((1, F, L), lambda i, te_r: (te_r[i, 0], 0, 0)),
        ],
        out_specs=pl.BlockSpec((_TILE, L), lambda i, te_r: (i, 0)),
    )
    eo = pl.pallas_call(
        _expert_kernel,
        grid_spec=grid_spec,
        out_shape=jax.ShapeDtypeStruct((PAD, L), jnp.float32),
        compiler_params=pltpu.CompilerParams(
            dimension_semantics=("arbitrary",)),
    )(te, zg, gate_up_w, down_w)

    r0, r1 = _sc_combine(eo, pos0, pos1)

    out = pl.pallas_call(
        _final_kernel,
        grid=(fgrid,),
        in_specs=[
            pl.BlockSpec((fblk, D), lambda i: (i, 0)),
            pl.BlockSpec((fblk, L), lambda i: (i, 0)),
            pl.BlockSpec((fblk, L), lambda i: (i, 0)),
            pl.BlockSpec((fblk, 2), lambda i: (i, 0)),
            pl.BlockSpec((D, 2 * Fs), lambda i: (0, 0)),
            pl.BlockSpec((Fs, D), lambda i: (0, 0)),
            pl.BlockSpec((L, D), lambda i: (0, 0)),
        ],
        out_specs=pl.BlockSpec((fblk, D), lambda i: (i, 0)),
        out_shape=jax.ShapeDtypeStruct((T, D), jnp.float32),
        scratch_shapes=[pltpu.VMEM((D, 2 * Fs), jnp.bfloat16),
                        pltpu.VMEM((Fs, D), jnp.bfloat16),
                        pltpu.VMEM((L, D), jnp.bfloat16)],
    )(x_flat, r0, r1, gating, shared_gu_w, shared_down_w, W_up)

    return (out.reshape(orig_shape), selected, gating, z_loss)


# speedup vs baseline: 1.0701x; 1.0054x over previous
import functools

import jax
import jax.numpy as jnp
from jax import lax
from jax.experimental import pallas as pl
from jax.experimental.pallas import tpu as pltpu
from jax.experimental.pallas import tpu_sc as plsc

_ZLOSS_COEF = 1e-3
_NEG = -1e30
_TILE = 256
_GW = 128


def _router_kernel(x_ref, wd_ref, gw_ref, bias_ref,
                   z_ref, sel_ref, gat_ref, pos_ref, te_ref, zl_ref,
                   carry_ref, csel_ref):
    i = pl.program_id(0)
    n = pl.num_programs(0)
    blk = x_ref.shape[0]
    row0 = i * blk
    x = x_ref[...]
    z = jnp.dot(x, wd_ref[...], preferred_element_type=jnp.float32)
    z_ref[...] = z
    logits = jnp.dot(z, gw_ref[...], preferred_element_type=jnp.float32)
    m = jnp.max(logits, axis=1, keepdims=True)
    lse = m[:, 0] + jnp.log(jnp.sum(jnp.exp(logits - m), axis=1))
    part = jnp.sum(lse * lse)

    @pl.when(i == 0)
    def _():
        zl_ref[...] = part.reshape(1, 1)

    @pl.when(i != 0)
    def _():
        zl_ref[...] += part.reshape(1, 1)

    aff = jax.nn.sigmoid(logits)
    scores = aff + bias_ref[...]
    E = scores.shape[1]
    iota = lax.broadcasted_iota(jnp.int32, scores.shape, 1)
    m1 = jnp.max(scores, axis=1, keepdims=True)
    a1 = jnp.min(jnp.where(scores == m1, iota, E), axis=1)
    oh1 = iota == a1[:, None]
    scores2 = jnp.where(oh1, _NEG, scores)
    m2 = jnp.max(scores2, axis=1, keepdims=True)
    a2 = jnp.min(jnp.where(scores2 == m2, iota, E), axis=1)
    oh2 = iota == a2[:, None]
    aff1 = jnp.sum(jnp.where(oh1, aff, 0.0), axis=1)
    aff2 = jnp.sum(jnp.where(oh2, aff, 0.0), axis=1)
    denom = aff1 + aff2 + 1e-9
    g1 = aff1 / denom
    g2 = aff2 / denom
    sel_ref[pl.ds(row0, blk), :] = jnp.concatenate(
        [a1[:, None], a2[:, None]], axis=1)
    gat_ref[...] = jnp.concatenate([g1[:, None], g2[:, None]], axis=1)

    A = oh1.astype(jnp.float32) + oh2.astype(jnp.float32)
    r = lax.broadcasted_iota(jnp.int32, (blk, blk), 0)
    c = lax.broadcasted_iota(jnp.int32, (blk, blk), 1)
    tril = (r >= c).astype(jnp.float32)
    cin = jnp.dot(tril, A, preferred_element_type=jnp.float32)

    @pl.when(i == 0)
    def _():
        carry_ref[...] = jnp.zeros_like(carry_ref)

    carry = carry_ref[...]
    cex = cin - A + carry
    csel0 = jnp.sum(jnp.where(oh1, cex, 0.0), axis=1)
    csel1 = jnp.sum(jnp.where(oh2, cex, 0.0), axis=1)
    csel_ref[pl.ds(row0, blk), :] = jnp.concatenate(
        [csel0[:, None], csel1[:, None]], axis=1)
    newcarry = carry + jnp.sum(A, axis=0, keepdims=True)
    carry_ref[...] = newcarry

    @pl.when(i == n - 1)
    def _():
        counts = newcarry
        pc = jnp.floor((counts + (_TILE - 1.0))
                       * (1.0 / _TILE)) * _TILE
        rr = lax.broadcasted_iota(jnp.int32, (E, E), 0)
        cc = lax.broadcasted_iota(jnp.int32, (E, E), 1)
        U = (rr < cc).astype(jnp.float32)
        pc8 = jnp.broadcast_to(pc, (8, E))
        cumx = jnp.dot(pc8, U, preferred_element_type=jnp.float32,
                       precision=jax.lax.Precision.HIGHEST)[0:1]
        cumi = cumx + pc
        sel_all = sel_ref[...]
        csel_all = csel_ref[...]
        T = sel_all.shape[0]
        eio = lax.broadcasted_iota(jnp.int32, (T, E), 1)
        cumxB = jnp.broadcast_to(cumx, (T, E))
        p0 = jnp.sum(jnp.where(eio == sel_all[:, 0][:, None], cumxB, 0.0),
                     axis=1) + csel_all[:, 0]
        p1 = jnp.sum(jnp.where(eio == sel_all[:, 1][:, None], cumxB, 0.0),
                     axis=1) + csel_all[:, 1]
        pos_ref[...] = jnp.concatenate(
            [p0[:, None], p1[:, None]], axis=1).astype(jnp.int32)

        TE_ROWS = te_ref.shape[0]
        NT = TE_ROWS - 16
        r64 = lax.broadcasted_iota(jnp.int32, (TE_ROWS, 1), 0)
        tstart = r64.astype(jnp.float32) * float(_TILE)
        cumiB = jnp.broadcast_to(cumi, (TE_ROWS, E))
        te_v = jnp.sum((cumiB <= tstart).astype(jnp.float32), axis=1,
                       keepdims=True)
        te_v = jnp.minimum(te_v, float(E - 1))
        n_act = cumi[:, E - 1:E] * (1.0 / _TILE)
        te_v = jnp.where(r64 == NT, jnp.broadcast_to(n_act, (TE_ROWS, 1)),
                         te_v)
        te_ref[...] = te_v.astype(jnp.int32)


def _expert_kernel(te_ref, zg_ref, guw_ref, dw_ref, eo_ref):
    i = pl.program_id(0)
    n_act = te_ref[te_ref.shape[0] - 16, 0]

    @pl.when(i < n_act)
    def _():
        zt = zg_ref[...].astype(jnp.bfloat16)
        h = jnp.dot(zt, guw_ref[0].astype(jnp.bfloat16),
                    preferred_element_type=jnp.float32)
        F = h.shape[1] // 2
        hh = (jax.nn.silu(h[:, :F]) * h[:, F:]).astype(jnp.bfloat16)
        eo = jnp.dot(hh, dw_ref[0].astype(jnp.bfloat16),
                     preferred_element_type=jnp.float32)
        eo_ref[...] = eo


def _final_kernel(x_ref, r0_ref, r1_ref, gat_ref, sguw_ref, sdw_ref, wup_ref,
                  out_ref, sguw16_ref, sdw16_ref, wup16_ref):
    @pl.when(pl.program_id(0) == 0)
    def _():
        sguw16_ref[...] = sguw_ref[...].astype(jnp.bfloat16)
        sdw16_ref[...] = sdw_ref[...].astype(jnp.bfloat16)
        wup16_ref[...] = wup_ref[...].astype(jnp.bfloat16)

    x = x_ref[...].astype(jnp.bfloat16)
    Fs = sdw_ref.shape[0]
    CH = 512
    g0 = gat_ref[:, 0:1]
    g1 = gat_ref[:, 1:2]
    ol = (g0 * r0_ref[...] + g1 * r1_ref[...]).astype(jnp.bfloat16)
    acc = jnp.dot(ol, wup16_ref[...], preferred_element_type=jnp.float32)
    for j in range(Fs // CH):
        gj = jnp.dot(x, sguw16_ref[:, j * CH:(j + 1) * CH],
                     preferred_element_type=jnp.float32)
        uj = jnp.dot(x, sguw16_ref[:, Fs + j * CH:Fs + (j + 1) * CH],
                     preferred_element_type=jnp.float32)
        hh = (jax.nn.silu(gj) * uj).astype(jnp.bfloat16)
        acc = acc + jnp.dot(hh, sdw16_ref[j * CH:(j + 1) * CH, :],
                            preferred_element_type=jnp.float32)
    out_ref[...] = acc


def _sc_dispatch(zl_, pos0, pos1, pad_rows):
    T, L = zl_.shape
    mesh = plsc.VectorSubcoreMesh(core_axis_name="c", subcore_axis_name="s")
    info = plsc.get_sparse_core_info()
    NC, NS = info.num_cores, info.num_subcores
    chunk = T // (NC * NS)

    @functools.partial(
        pl.kernel, mesh=mesh,
        out_type=jax.ShapeDtypeStruct((pad_rows, L), zl_.dtype),
        scratch_types=[
            pltpu.VMEM((chunk,), jnp.int32),
            pltpu.VMEM((chunk,), jnp.int32),
            pltpu.VMEM((chunk, L), zl_.dtype),
            pltpu.SemaphoreType.DMA,
            pltpu.SemaphoreType.DMA,
        ])
    def k(z_hbm, p0_hbm, p1_hbm, zg_hbm, p0_v, p1_v, rows_v, sem1, sem2):
        wid = lax.axis_index("s") * NC + lax.axis_index("c")
        base = wid * chunk
        pltpu.sync_copy(p0_hbm.at[pl.ds(base, chunk)], p0_v)
        pltpu.sync_copy(p1_hbm.at[pl.ds(base, chunk)], p1_v)
        pltpu.sync_copy(z_hbm.at[pl.ds(base, chunk)], rows_v)
        c1 = pltpu.async_copy(rows_v, zg_hbm.at[p0_v], sem1)
        c2 = pltpu.async_copy(rows_v, zg_hbm.at[p1_v], sem2)
        c1.wait()
        c2.wait()

    return k(zl_, pos0, pos1)


def _sc_combine(eo_, pos0, pos1):
    _, L = eo_.shape
    T = pos0.shape[0]
    mesh = plsc.VectorSubcoreMesh(core_axis_name="c", subcore_axis_name="s")
    info = plsc.get_sparse_core_info()
    NC, NS = info.num_cores, info.num_subcores
    chunk = T // (NC * NS)

    @functools.partial(
        pl.kernel, mesh=mesh,
        out_type=(jax.ShapeDtypeStruct((T, L), eo_.dtype),
                  jax.ShapeDtypeStruct((T, L), eo_.dtype)),
        scratch_types=[
            pltpu.VMEM((chunk,), jnp.int32),
            pltpu.VMEM((chunk,), jnp.int32),
            pltpu.VMEM((chunk, L), eo_.dtype),
            pltpu.VMEM((chunk, L), eo_.dtype),
            pltpu.SemaphoreType.DMA,
            pltpu.SemaphoreType.DMA,
        ])
    def k(eo_hbm, p0_hbm, p1_hbm, r0_hbm, r1_hbm,
          p0_v, p1_v, r0_v, r1_v, sem1, sem2):
        wid = lax.axis_index("s") * NC + lax.axis_index("c")
        base = wid * chunk
        pltpu.sync_copy(p0_hbm.at[pl.ds(base, chunk)], p0_v)
        pltpu.sync_copy(p1_hbm.at[pl.ds(base, chunk)], p1_v)
        c1 = pltpu.async_copy(eo_hbm.at[p0_v], r0_v, sem1)
        c2 = pltpu.async_copy(eo_hbm.at[p1_v], r1_v, sem2)
        c1.wait()
        c2.wait()
        pltpu.sync_copy(r0_v, r0_hbm.at[pl.ds(base, chunk)])
        pltpu.sync_copy(r1_v, r1_hbm.at[pl.ds(base, chunk)])

    return k(eo_, pos0, pos1)


def kernel(x, W_down, gate_w, expert_bias, gate_up_w, down_w, W_up,
           shared_gu_w, shared_down_w):
    orig_shape = x.shape
    D = x.shape[-1]
    x_flat = x.reshape(-1, D)
    T = x_flat.shape[0]
    L = W_down.shape[1]
    E = gate_w.shape[1]
    F = down_w.shape[1]
    Fs = shared_down_w.shape[0]
    TOPK = 2
    NP = T * TOPK
    PAD = ((NP + E * (_TILE - 1) + _TILE - 1) // _TILE) * _TILE
    NT = PAD // _TILE

    blk = 512
    grid_t = T // blk
    fblk = 512
    fgrid = T // fblk

    zrt, selected, gating, pos, te, zl = pl.pallas_call(
        _router_kernel,
        grid=(grid_t,),
        in_specs=[
            pl.BlockSpec((blk, D), lambda i: (i, 0)),
            pl.BlockSpec((D, L), lambda i: (0, 0)),
            pl.BlockSpec((L, E), lambda i: (0, 0)),
            pl.BlockSpec((1, E), lambda i: (0, 0)),
        ],
        out_specs=[
            pl.BlockSpec((blk, L), lambda i: (i, 0)),
            pl.BlockSpec((T, 2), lambda i: (0, 0)),
            pl.BlockSpec((blk, 2), lambda i: (i, 0)),
            pl.BlockSpec((T, 2), lambda i: (0, 0)),
            pl.BlockSpec((NT + 16, 1), lambda i: (0, 0)),
            pl.BlockSpec((1, 1), lambda i: (0, 0)),
        ],
        out_shape=[
            jax.ShapeDtypeStruct((T, L), jnp.float32),
            jax.ShapeDtypeStruct((T, 2), jnp.int32),
            jax.ShapeDtypeStruct((T, 2), jnp.float32),
            jax.ShapeDtypeStruct((T, 2), jnp.int32),
            jax.ShapeDtypeStruct((NT + 16, 1), jnp.int32),
            jax.ShapeDtypeStruct((1, 1), jnp.float32),
        ],
        scratch_shapes=[pltpu.VMEM((1, E), jnp.float32),
                        pltpu.VMEM((T, 2), jnp.float32)],
    )(x_flat, W_down, gate_w, expert_bias.reshape(1, E))

    z_loss = (_ZLOSS_COEF / T) * zl[0, 0]

    pos0 = pos[:, 0]
    pos1 = pos[:, 1]

    zg = _sc_dispatch(zrt, pos0, pos1, PAD)

    grid_spec = pltpu.PrefetchScalarGridSpec(
        num_scalar_prefetch=1,
        grid=(NT,),
        in_specs=[
            pl.BlockSpec((_TILE, L), lambda i, te_r: (i, 0)),
            pl.BlockSpec((1, L, 2 * F), lambda i, te_r: (te_r[i, 0], 0, 0)),
            pl.BlockSpec((1, F, L), lambda i, te_r: (te_r[i, 0], 0, 0)),
        ],
        out_specs=pl.BlockSpec((_TILE, L), lambda i, te_r: (i, 0)),
    )
    eo = pl.pallas_call(
        _expert_kernel,
        grid_spec=grid_spec,
        out_shape=jax.ShapeDtypeStruct((PAD, L), jnp.float32),
        compiler_params=pltpu.CompilerParams(
            dimension_semantics=("arbitrary",)),
    )(te, zg, gate_up_w, down_w)

    r0, r1 = _sc_combine(eo, pos0, pos1)

    out = pl.pallas_call(
        _final_kernel,
        grid=(fgrid,),
        in_specs=[
            pl.BlockSpec((fblk, D), lambda i: (i, 0)),
            pl.BlockSpec((fblk, L), lambda i: (i, 0)),
            pl.BlockSpec((fblk, L), lambda i: (i, 0)),
            pl.BlockSpec((fblk, 2), lambda i: (i, 0)),
            pl.BlockSpec((D, 2 * Fs), lambda i: (0, 0)),
            pl.BlockSpec((Fs, D), lambda i: (0, 0)),
            pl.BlockSpec((L, D), lambda i: (0, 0)),
        ],
        out_specs=pl.BlockSpec((fblk, D), lambda i: (i, 0)),
        out_shape=jax.ShapeDtypeStruct((T, D), jnp.float32),
        scratch_shapes=[pltpu.VMEM((D, 2 * Fs), jnp.bfloat16),
                        pltpu.VMEM((Fs, D), jnp.bfloat16),
                        pltpu.VMEM((L, D), jnp.bfloat16)],
    )(x_flat, r0, r1, gating, shared_gu_w, shared_down_w, W_up)

    return (out.reshape(orig_shape), selected, gating, z_loss)
